# baseline (speedup 1.0000x reference)
import functools

import jax
import jax.numpy as jnp
from jax import lax
from jax.experimental import pallas as pl
from jax.experimental.pallas import tpu as pltpu
from jax.experimental.pallas import tpu_sc as plsc

_H = 8
_D = 64
_NCAP = 512
_BQ = 256
_BK = 256
_BIG = jnp.int32(1 << 30)

_SC_INFO = plsc.get_sparse_core_info()
_NC = _SC_INFO.num_cores
_NS = _SC_INFO.num_subcores
_L = _SC_INFO.num_lanes
_NW = _NC * _NS


def _seg_start_scalar(sref, nseg, pos):
    seg = jnp.int32(0)
    for b in range(nseg):
        ob = sref[b]
        seg = jnp.where(ob <= pos, jnp.maximum(seg, ob), seg)
    return seg


def _sc_seg_body(starts_hbm, out_hbm, off_v, seg_v, *, nseg, tok_per_w):
    wid = lax.axis_index("s") * _NC + lax.axis_index("c")
    base = wid * tok_per_w
    pltpu.sync_copy(starts_hbm, off_v)
    obs = [off_v[b] for b in range(nseg)]
    for v in range(tok_per_w // _L):
        t = base + v * _L + lax.iota(jnp.int32, _L)
        seg = jnp.zeros((_L,), jnp.int32)
        for ob in obs:
            seg = jnp.where(ob <= t, jnp.maximum(seg, ob), seg)
        seg = jnp.where(t - seg < _NCAP, seg, _BIG)
        seg_v[pl.ds(v * _L, _L)] = seg
    pltpu.sync_copy(seg_v, out_hbm.at[pl.ds(base, tok_per_w)])


def _seg_starts_sc(starts, T, nseg):
    tok_per_w = T // _NW
    mesh = plsc.VectorSubcoreMesh(core_axis_name="c", subcore_axis_name="s")
    return pl.kernel(
        functools.partial(_sc_seg_body, nseg=nseg, tok_per_w=tok_per_w),
        mesh=mesh,
        out_type=jax.ShapeDtypeStruct((T,), jnp.int32),
        scratch_types=[
            pltpu.VMEM((nseg, _L), jnp.int32),
            pltpu.VMEM((tok_per_w,), jnp.int32),
        ],
    )(starts)


def _hstu_body(sref, q_ref, k_ref, v_ref, seg_ref, o_ref, *, nseg):
    i = pl.program_id(0)
    t0 = i * _BQ

    s0 = _seg_start_scalar(sref, nseg, t0)
    s_last = _seg_start_scalar(sref, nseg, t0 + _BQ - 1)
    skip = (s_last == s0) & (t0 - s0 >= _NCAP)

    @pl.when(skip)
    def _():
        o_ref[:, :] = jnp.zeros_like(o_ref)

    @pl.when(jnp.logical_not(skip))
    def _():
        seg_q = seg_ref[:, :]

        scale = 1.0 / sref[nseg + 1].astype(jnp.float32)
        qh = [q_ref[:, h * _D:(h + 1) * _D] for h in range(_H)]

        def pair(j, accs, extra_mask):
            kb = k_ref[pl.ds(j * _BK, _BK), :]
            vb = v_ref[pl.ds(j * _BK, _BK), :]
            cols = j * _BK + jax.lax.broadcasted_iota(jnp.int32, (1, _BK), 1)
            mask = cols >= seg_q
            if extra_mask is not None:
                mask = mask & extra_mask
            new = []
            for h in range(_H):
                s = jax.lax.dot_general(
                    qh[h], kb[:, h * _D:(h + 1) * _D],
                    (((1,), (1,)), ((), ())),
                    preferred_element_type=jnp.float32)
                p = jnp.where(mask, s * jax.nn.sigmoid(s), 0.0)
                new.append(accs[h] + jax.lax.dot_general(
                    p, vb[:, h * _D:(h + 1) * _D],
                    (((1,), (0,)), ((), ())),
                    preferred_element_type=jnp.float32))
            return tuple(new)

        jmin = jnp.maximum(s0, t0 - (_NCAP - 1)) // _BK
        init = tuple(jnp.zeros((_BQ, _D), jnp.float32) for _ in range(_H))
        accs = jax.lax.fori_loop(
            jmin, i, lambda j, a: pair(j, a, None), init)
        causal = (jax.lax.broadcasted_iota(jnp.int32, (_BQ, _BK), 0)
                  >= jax.lax.broadcasted_iota(jnp.int32, (_BQ, _BK), 1))
        accs = pair(i, accs, causal)
        o_ref[:, :] = jnp.concatenate(accs, axis=1) * scale


def kernel(tq, tk, tv, offsets, max_seqlen):
    T, dqk = tq.shape
    dv = tv.shape[1]
    nseg = offsets.shape[0] - 1
    starts = jnp.broadcast_to(
        offsets[:-1].astype(jnp.int32)[:, None], (nseg, _L))
    seg_eff = _seg_starts_sc(starts, T, nseg).reshape(T, 1)
    scalars = jnp.concatenate([
        offsets.astype(jnp.int32),
        jnp.asarray(max_seqlen, jnp.int32).reshape(1),
    ])
    grid = (T // _BQ,)
    out = pl.pallas_call(
        functools.partial(_hstu_body, nseg=nseg),
        grid_spec=pltpu.PrefetchScalarGridSpec(
            num_scalar_prefetch=1,
            grid=grid,
            in_specs=[
                pl.BlockSpec((_BQ, dqk), lambda i, s: (i, 0)),
                pl.BlockSpec((T, dqk), lambda i, s: (0, 0)),
                pl.BlockSpec((T, dv), lambda i, s: (0, 0)),
                pl.BlockSpec((_BQ, 1), lambda i, s: (i, 0)),
            ],
            out_specs=pl.BlockSpec((_BQ, dv), lambda i, s: (i, 0)),
        ),
        compiler_params=pltpu.CompilerParams(
            dimension_semantics=("parallel",)),
        out_shape=jax.ShapeDtypeStruct((T, dv), tq.dtype),
    )(scalars, tq, tk, tv, seg_eff)
    return out


# speedup vs baseline: 1.4277x; 1.4277x over previous
import functools

import jax
import jax.numpy as jnp
from jax.experimental import pallas as pl
from jax.experimental.pallas import tpu as pltpu

_H = 8
_D = 64
_NCAP = 512
_BQ = 256
_BK = 256
_BIG = 1 << 30


def _seg_start_scalar(sref, nseg, pos):
    seg = jnp.int32(0)
    for b in range(nseg):
        ob = sref[b]
        seg = jnp.where(ob <= pos, jnp.maximum(seg, ob), seg)
    return seg


def _hstu_body(sref, q_ref, k_ref, v_ref, o_ref, seg_ref, *, nseg):
    i = pl.program_id(0)
    t0 = i * _BQ
    T = k_ref.shape[0]

    @pl.when(i == 0)
    def _():
        toks = jax.lax.broadcasted_iota(jnp.int32, (1, T), 1)
        seg = jnp.zeros((1, T), jnp.int32)
        for b in range(nseg):
            ob = sref[b]
            seg = jnp.where(ob <= toks, jnp.maximum(seg, ob), seg)
        seg_ref[:, :] = jnp.where(toks - seg < _NCAP, seg, _BIG)

    s0 = _seg_start_scalar(sref, nseg, t0)
    s_last = _seg_start_scalar(sref, nseg, t0 + _BQ - 1)
    skip = (s_last == s0) & (t0 - s0 >= _NCAP)

    @pl.when(skip)
    def _():
        o_ref[:, :] = jnp.zeros_like(o_ref)

    @pl.when(jnp.logical_not(skip))
    def _():
        seg_q = seg_ref[:, pl.ds(t0, _BQ)].reshape(_BQ, 1)

        scale = 1.0 / sref[nseg + 1].astype(jnp.float32)
        qh = [q_ref[:, h * _D:(h + 1) * _D] for h in range(_H)]

        def pair(j, accs, extra_mask):
            kb = k_ref[pl.ds(j * _BK, _BK), :]
            vb = v_ref[pl.ds(j * _BK, _BK), :]
            cols = j * _BK + jax.lax.broadcasted_iota(jnp.int32, (1, _BK), 1)
            mask = cols >= seg_q
            if extra_mask is not None:
                mask = mask & extra_mask
            new = []
            for h in range(_H):
                s = jax.lax.dot_general(
                    qh[h], kb[:, h * _D:(h + 1) * _D],
                    (((1,), (1,)), ((), ())),
                    preferred_element_type=jnp.float32)
                p = jnp.where(mask, s * jax.nn.sigmoid(s), 0.0)
                new.append(accs[h] + jax.lax.dot_general(
                    p, vb[:, h * _D:(h + 1) * _D],
                    (((1,), (0,)), ((), ())),
                    preferred_element_type=jnp.float32))
            return tuple(new)

        jmin = jnp.maximum(s0, t0 - (_NCAP - 1)) // _BK
        init = tuple(jnp.zeros((_BQ, _D), jnp.float32) for _ in range(_H))
        accs = jax.lax.fori_loop(
            jmin, i, lambda j, a: pair(j, a, None), init)
        causal = (jax.lax.broadcasted_iota(jnp.int32, (_BQ, _BK), 0)
                  >= jax.lax.broadcasted_iota(jnp.int32, (_BQ, _BK), 1))
        accs = pair(i, accs, causal)
        o_ref[:, :] = jnp.concatenate(accs, axis=1) * scale


def kernel(tq, tk, tv, offsets, max_seqlen):
    T, dqk = tq.shape
    dv = tv.shape[1]
    nseg = offsets.shape[0] - 1
    scalars = jnp.concatenate([
        offsets.astype(jnp.int32),
        jnp.asarray(max_seqlen, jnp.int32).reshape(1),
    ])
    grid = (T // _BQ,)
    out = pl.pallas_call(
        functools.partial(_hstu_body, nseg=nseg),
        grid_spec=pltpu.PrefetchScalarGridSpec(
            num_scalar_prefetch=1,
            grid=grid,
            in_specs=[
                pl.BlockSpec((_BQ, dqk), lambda i, s: (i, 0)),
                pl.BlockSpec((T, dqk), lambda i, s: (0, 0)),
                pl.BlockSpec((T, dv), lambda i, s: (0, 0)),
            ],
            out_specs=pl.BlockSpec((_BQ, dv), lambda i, s: (i, 0)),
            scratch_shapes=[pltpu.VMEM((1, T), jnp.int32)],
        ),
        compiler_params=pltpu.CompilerParams(
            dimension_semantics=("arbitrary",)),
        out_shape=jax.ShapeDtypeStruct((T, dv), tq.dtype),
    )(scalars, tq, tk, tv)
    return out
